# baseline (speedup 1.0000x reference)
import functools

import jax
import jax.numpy as jnp
from jax import lax
from jax.experimental import pallas as pl
from jax.experimental.pallas import tpu as pltpu
from jax.experimental.pallas import tpu_sc as plsc

N = 10000
E = 320000
D = 128
DE = 4

NC = 2
NS = 16
NW = NC * NS
EPT = E // NW
CHUNK = 40
NCH = EPT // CHUNK
ROWS_PT = 624
RB = 1000
EB = 8000
HD = D // 2

FP_SCALE = 1024.0
FP_BIAS = 8192


def _pack_pairs(p):
    q = jnp.clip(jnp.round(p * FP_SCALE), -8191.0, 8191.0) + float(FP_BIAS)
    qi = q.astype(jnp.int32)
    return (qi[:, HD:] << 16) | qi[:, :HD]


def _proj_body(x_ref, w1_ref, w2_ref, b_ref, t_ref):
    xb = x_ref[...]
    t_ref[...] = jnp.concatenate(
        [_pack_pairs(jnp.dot(xb, w1_ref[...],
                             preferred_element_type=jnp.float32)
                     + b_ref[...][None, :]),
         _pack_pairs(jnp.dot(xb, w2_ref[...],
                             preferred_element_type=jnp.float32))], axis=1)


def _edge_term_body(ea_ref, w3_ref, c_ref):
    c_ref[...] = _pack_pairs(jnp.dot(ea_ref[...], w3_ref[0:DE, :],
                                     preferred_element_type=jnp.float32))


def _update_body(x_ref, p_ref, wu1_ref, wu2_ref, b_ref, o_ref):
    xb = x_ref[...]
    agg = p_ref[0] + p_ref[1]
    h = (jnp.dot(xb, wu1_ref[...], preferred_element_type=jnp.float32)
         + jnp.dot(agg, wu2_ref[...], preferred_element_type=jnp.float32)
         + b_ref[...][None, :])
    o_ref[...] = xb + jnp.maximum(h, 0.0)


def _sc_edges_body(t_hbm, src_hbm, dst_hbm, c_hbm, out_hbm,
                   i0, i1, i2, i3, a0, b0, c0, m0, a1, b1, c1, m1, agg,
                   sg0, sg1, si0, si1, si2, si3, ss0, ss1):
    cid = lax.axis_index("c")
    sid = lax.axis_index("s")
    wid = cid * NS + sid

    def _zrow(r, carry):
        for d in range(D // 16):
            m0[r, pl.ds(d * 16, 16)] = jnp.zeros((16,), jnp.float32)
        return carry
    lax.fori_loop(0, CHUNK, _zrow, 0)
    base = sid * ROWS_PT
    for k in range(ROWS_PT // CHUNK):
        pltpu.sync_copy(m0, agg.at[pl.ds(base + k * CHUNK, CHUNK)])
    rem = ROWS_PT % CHUNK
    if rem:
        pltpu.sync_copy(m0.at[pl.ds(0, rem)],
                        agg.at[pl.ds(base + ROWS_PT - rem, rem)])

    @pl.when(sid == NS - 1)
    def _zero_tail():
        pltpu.sync_copy(m0.at[pl.ds(0, N - NS * ROWS_PT)],
                        agg.at[pl.ds(NS * ROWS_PT, N - NS * ROWS_PT)])
    plsc.subcore_barrier()

    lomask = jnp.int32(0xFFFF)
    inv_scale = 1.0 / FP_SCALE
    bias3 = 3.0 * FP_BIAS / FP_SCALE

    IDX = (i0, i1, i2, i3)
    SI = (si0, si1, si2, si3)
    AB = ((a0, b0, c0, m0), (a1, b1, c1, m1))
    SG = (sg0, sg1)
    SS = (ss0, ss1)
    ibase = wid * EPT

    def _process(s):
        bufA, bufB, bufC, M = AB[s]

        def _edge(e, carry):
            for t in range(HD // 16):
                s3 = (bufA[e, pl.ds(16 * t, 16)]
                      + bufB[e, pl.ds(HD + 16 * t, 16)]
                      + bufC[e, pl.ds(16 * t, 16)])
                lo = (s3 & lomask).astype(jnp.float32) * inv_scale - bias3
                hi = (((s3 >> 16) & lomask).astype(jnp.float32)
                      * inv_scale - bias3)
                M[e, pl.ds(16 * t, 16)] = jnp.maximum(lo, 0.0)
                M[e, pl.ds(HD + 16 * t, 16)] = jnp.maximum(hi, 0.0)
            return carry
        lax.fori_loop(0, CHUNK, _edge, 0)

    def _idx_issue(j, r):
        pltpu.async_copy(src_hbm.at[pl.ds(ibase + j * CHUNK, CHUNK)],
                         IDX[r].at[0], SI[r])
        pltpu.async_copy(dst_hbm.at[pl.ds(ibase + j * CHUNK, CHUNK)],
                         IDX[r].at[1], SI[r])

    def _idx_drain(r):
        pltpu.make_async_copy(src_hbm.at[pl.ds(0, CHUNK)],
                              IDX[r].at[0], SI[r]).wait()
        pltpu.make_async_copy(src_hbm.at[pl.ds(0, CHUNK)],
                              IDX[r].at[1], SI[r]).wait()

    def _gathers(j, s, r):
        bufA, bufB, bufC, _ = AB[s]
        pltpu.async_copy(t_hbm.at[IDX[r].at[0]], bufA, SG[s])
        pltpu.async_copy(t_hbm.at[IDX[r].at[1]], bufB, SG[s])
        pltpu.async_copy(c_hbm.at[wid * NCH + j], bufC, SG[s])

    def _gdrain(s, r):
        bufA, bufB, bufC, _ = AB[s]
        pltpu.make_async_copy(t_hbm.at[IDX[r].at[0]], bufA, SG[s]).wait()
        pltpu.make_async_copy(t_hbm.at[IDX[r].at[1]], bufB, SG[s]).wait()
        pltpu.make_async_copy(c_hbm.at[0], bufC, SG[s]).wait()

    def _sc_issue(s, r):
        pltpu.async_copy(AB[s][3], agg.at[IDX[r].at[1]], SS[s], add=True)

    def _sc_drain(s, r):
        pltpu.make_async_copy(AB[s][3], agg.at[IDX[r].at[1]],
                              SS[s]).wait()

    def _handle(j, s, r, wait_sc, prefetch_idx, issue_next):
        _gdrain(s, r)
        if issue_next:
            if wait_sc:
                _sc_drain(1 - s, (r + 3) % 4)
            _idx_drain((r + 1) % 4)
            _gathers(j + 1, 1 - s, (r + 1) % 4)
        _process(s)
        _sc_issue(s, r)
        if prefetch_idx:
            _idx_issue(j + 2, (r + 2) % 4)

    _idx_issue(0, 0)
    _idx_drain(0)
    _gathers(0, 0, 0)
    _idx_issue(1, 1)
    _handle(0, 0, 0, False, True, True)
    _handle(1, 1, 1, True, True, True)

    def _quad(i, carry):
        jb = 2 + 4 * i
        _handle(jb, 0, 2, True, True, True)
        _handle(jb + 1, 1, 3, True, True, True)
        _handle(jb + 2, 0, 0, True, True, True)
        _handle(jb + 3, 1, 1, True, True, True)
        return carry
    lax.fori_loop(0, (NCH - 6) // 4, _quad, 0)

    _handle(NCH - 4, 0, 2, True, True, True)
    _handle(NCH - 3, 1, 3, True, True, True)
    _handle(NCH - 2, 0, 0, True, False, True)
    _handle(NCH - 1, 1, 1, True, False, False)
    _sc_drain(0, 0)
    _sc_drain(1, 1)

    plsc.subcore_barrier()
    pltpu.sync_copy(agg.at[pl.ds(base, ROWS_PT)],
                    out_hbm.at[cid, pl.ds(base, ROWS_PT)])

    @pl.when(sid == NS - 1)
    def _flush_tail():
        pltpu.sync_copy(agg.at[pl.ds(NS * ROWS_PT, N - NS * ROWS_PT)],
                        out_hbm.at[cid, pl.ds(NS * ROWS_PT, N - NS * ROWS_PT)])


_sc_edges = functools.partial(
    pl.kernel,
    out_type=jax.ShapeDtypeStruct((NC, N, D), jnp.float32),
    mesh=plsc.VectorSubcoreMesh(core_axis_name="c", subcore_axis_name="s"),
    scratch_types=[
        pltpu.VMEM((2, CHUNK), jnp.int32),
        pltpu.VMEM((2, CHUNK), jnp.int32),
        pltpu.VMEM((2, CHUNK), jnp.int32),
        pltpu.VMEM((2, CHUNK), jnp.int32),
        pltpu.VMEM((CHUNK, D), jnp.int32),
        pltpu.VMEM((CHUNK, D), jnp.int32),
        pltpu.VMEM((CHUNK, HD), jnp.int32),
        pltpu.VMEM((CHUNK, D), jnp.float32),
        pltpu.VMEM((CHUNK, D), jnp.int32),
        pltpu.VMEM((CHUNK, D), jnp.int32),
        pltpu.VMEM((CHUNK, HD), jnp.int32),
        pltpu.VMEM((CHUNK, D), jnp.float32),
        pltpu.VMEM_SHARED((N, D), jnp.float32),
        pltpu.SemaphoreType.DMA,
        pltpu.SemaphoreType.DMA,
        pltpu.SemaphoreType.DMA,
        pltpu.SemaphoreType.DMA,
        pltpu.SemaphoreType.DMA,
        pltpu.SemaphoreType.DMA,
        pltpu.SemaphoreType.DMA,
        pltpu.SemaphoreType.DMA,
    ],
)(_sc_edges_body)


def kernel(x, edge_index, edge_attr, W_msg, b_msg, W_upd, b_upd):
    tnode = pl.pallas_call(
        _proj_body,
        grid=(N // RB,),
        in_specs=[
            pl.BlockSpec((RB, D), lambda i: (i, 0)),
            pl.BlockSpec((D, D), lambda i: (0, 0)),
            pl.BlockSpec((D, D), lambda i: (1, 0)),
            pl.BlockSpec((D,), lambda i: (0,)),
        ],
        out_specs=pl.BlockSpec((RB, D), lambda i: (i, 0)),
        out_shape=jax.ShapeDtypeStruct((N, D), jnp.int32),
    )(x, W_msg, W_msg, b_msg)

    c_edge = pl.pallas_call(
        _edge_term_body,
        grid=(E // EB,),
        in_specs=[
            pl.BlockSpec((EB, DE), lambda i: (i, 0)),
            pl.BlockSpec((8, D), lambda i: (2 * D // 8, 0)),
        ],
        out_specs=pl.BlockSpec((EB, HD), lambda i: (i, 0)),
        out_shape=jax.ShapeDtypeStruct((E, HD), jnp.int32),
    )(edge_attr, W_msg)

    parts = _sc_edges(tnode, edge_index[0], edge_index[1],
                      c_edge.reshape(NW * NCH, CHUNK, HD))

    out = pl.pallas_call(
        _update_body,
        grid=(N // RB,),
        in_specs=[
            pl.BlockSpec((RB, D), lambda i: (i, 0)),
            pl.BlockSpec((NC, RB, D), lambda i: (0, i, 0)),
            pl.BlockSpec((D, D), lambda i: (0, 0)),
            pl.BlockSpec((D, D), lambda i: (1, 0)),
            pl.BlockSpec((D,), lambda i: (0,)),
        ],
        out_specs=pl.BlockSpec((RB, D), lambda i: (i, 0)),
        out_shape=jax.ShapeDtypeStruct((N, D), jnp.float32),
    )(x, parts, W_upd, W_upd, b_upd)
    return out


# speedup vs baseline: 1.0150x; 1.0150x over previous
import functools

import jax
import jax.numpy as jnp
from jax import lax
from jax.experimental import pallas as pl
from jax.experimental.pallas import tpu as pltpu
from jax.experimental.pallas import tpu_sc as plsc

N = 10000
E = 320000
D = 128
DE = 4

NC = 2
NS = 16
NW = NC * NS
EPT = E // NW
CHUNK = 40
NCH = EPT // CHUNK
ROWS_PT = 624
RB = 1000
EB = 8000
EI = E
HD = D // 2

FP_SCALE = 1024.0
FP_BIAS = 8192


def _pack_pairs(p):
    q = jnp.clip(jnp.round(p * FP_SCALE), -8191.0, 8191.0) + float(FP_BIAS)
    qi = q.astype(jnp.int32)
    return (qi[:, HD:] << 16) | qi[:, :HD]


def _proj_body(x_ref, w1_ref, w2_ref, b_ref, t_ref):
    xb = x_ref[...]
    t_ref[...] = jnp.concatenate(
        [_pack_pairs(jnp.dot(xb, w1_ref[...],
                             preferred_element_type=jnp.float32)
                     + b_ref[...][None, :]),
         _pack_pairs(jnp.dot(xb, w2_ref[...],
                             preferred_element_type=jnp.float32))], axis=1)


def _edge_term_body(ea_ref, w3_ref, c_ref):
    c_ref[...] = _pack_pairs(jnp.dot(ea_ref[...], w3_ref[0:DE, :],
                                     preferred_element_type=jnp.float32))


def _split_body(ei_ref, s_ref, d_ref):
    s_ref[...] = ei_ref[0]
    d_ref[...] = ei_ref[1]


def _update_body(x_ref, p_ref, wu1_ref, wu2_ref, b_ref, o_ref):
    xb = x_ref[...]
    agg = p_ref[0] + p_ref[1]
    h = (jnp.dot(xb, wu1_ref[...], preferred_element_type=jnp.float32)
         + jnp.dot(agg, wu2_ref[...], preferred_element_type=jnp.float32)
         + b_ref[...][None, :])
    o_ref[...] = xb + jnp.maximum(h, 0.0)


def _sc_edges_body(t_hbm, src_hbm, dst_hbm, c_hbm, out_hbm,
                   i0, i1, i2, i3, a0, b0, c0, m0, a1, b1, c1, m1, agg,
                   sg0, sg1, si0, si1, si2, si3, ss0, ss1):
    cid = lax.axis_index("c")
    sid = lax.axis_index("s")
    wid = cid * NS + sid

    def _zrow(r, carry):
        for d in range(D // 16):
            m0[r, pl.ds(d * 16, 16)] = jnp.zeros((16,), jnp.float32)
        return carry
    lax.fori_loop(0, CHUNK, _zrow, 0)
    base = sid * ROWS_PT
    for k in range(ROWS_PT // CHUNK):
        pltpu.sync_copy(m0, agg.at[pl.ds(base + k * CHUNK, CHUNK)])
    rem = ROWS_PT % CHUNK
    if rem:
        pltpu.sync_copy(m0.at[pl.ds(0, rem)],
                        agg.at[pl.ds(base + ROWS_PT - rem, rem)])

    @pl.when(sid == NS - 1)
    def _zero_tail():
        pltpu.sync_copy(m0.at[pl.ds(0, N - NS * ROWS_PT)],
                        agg.at[pl.ds(NS * ROWS_PT, N - NS * ROWS_PT)])
    plsc.subcore_barrier()

    lomask = jnp.int32(0xFFFF)
    inv_scale = 1.0 / FP_SCALE
    bias3 = 3.0 * FP_BIAS / FP_SCALE

    IDX = (i0, i1, i2, i3)
    SI = (si0, si1, si2, si3)
    AB = ((a0, b0, c0, m0), (a1, b1, c1, m1))
    SG = (sg0, sg1)
    SS = (ss0, ss1)
    ibase = wid * EPT

    def _process(s):
        bufA, bufB, bufC, M = AB[s]

        def _edge(e, carry):
            for t in range(HD // 16):
                s3 = (bufA[e, pl.ds(16 * t, 16)]
                      + bufB[e, pl.ds(HD + 16 * t, 16)]
                      + bufC[e, pl.ds(16 * t, 16)])
                lo = (s3 & lomask).astype(jnp.float32) * inv_scale - bias3
                hi = (((s3 >> 16) & lomask).astype(jnp.float32)
                      * inv_scale - bias3)
                M[e, pl.ds(16 * t, 16)] = jnp.maximum(lo, 0.0)
                M[e, pl.ds(HD + 16 * t, 16)] = jnp.maximum(hi, 0.0)
            return carry
        lax.fori_loop(0, CHUNK, _edge, 0)

    def _idx_issue(j, r):
        pltpu.async_copy(src_hbm.at[pl.ds(ibase + j * CHUNK, CHUNK)],
                         IDX[r].at[0], SI[r])
        pltpu.async_copy(dst_hbm.at[pl.ds(ibase + j * CHUNK, CHUNK)],
                         IDX[r].at[1], SI[r])

    def _idx_drain(r):
        pltpu.make_async_copy(src_hbm.at[pl.ds(0, CHUNK)],
                              IDX[r].at[0], SI[r]).wait()
        pltpu.make_async_copy(src_hbm.at[pl.ds(0, CHUNK)],
                              IDX[r].at[1], SI[r]).wait()

    def _gathers(j, s, r):
        bufA, bufB, bufC, _ = AB[s]
        pltpu.async_copy(t_hbm.at[IDX[r].at[0]], bufA, SG[s])
        pltpu.async_copy(t_hbm.at[IDX[r].at[1]], bufB, SG[s])
        pltpu.async_copy(c_hbm.at[wid * NCH + j], bufC, SG[s])

    def _gdrain(s, r):
        bufA, bufB, bufC, _ = AB[s]
        pltpu.make_async_copy(t_hbm.at[IDX[r].at[0]], bufA, SG[s]).wait()
        pltpu.make_async_copy(t_hbm.at[IDX[r].at[1]], bufB, SG[s]).wait()
        pltpu.make_async_copy(c_hbm.at[0], bufC, SG[s]).wait()

    def _sc_issue(s, r):
        pltpu.async_copy(AB[s][3], agg.at[IDX[r].at[1]], SS[s], add=True)

    def _sc_drain(s, r):
        pltpu.make_async_copy(AB[s][3], agg.at[IDX[r].at[1]],
                              SS[s]).wait()

    def _handle(j, s, r, wait_sc, prefetch_idx, issue_next):
        _gdrain(s, r)
        if issue_next:
            if wait_sc:
                _sc_drain(1 - s, (r + 3) % 4)
            _idx_drain((r + 1) % 4)
            _gathers(j + 1, 1 - s, (r + 1) % 4)
        _process(s)
        _sc_issue(s, r)
        if prefetch_idx:
            _idx_issue(j + 2, (r + 2) % 4)

    _idx_issue(0, 0)
    _idx_drain(0)
    _gathers(0, 0, 0)
    _idx_issue(1, 1)
    _handle(0, 0, 0, False, True, True)
    _handle(1, 1, 1, True, True, True)

    def _quad(i, carry):
        jb = 2 + 4 * i
        _handle(jb, 0, 2, True, True, True)
        _handle(jb + 1, 1, 3, True, True, True)
        _handle(jb + 2, 0, 0, True, True, True)
        _handle(jb + 3, 1, 1, True, True, True)
        return carry
    lax.fori_loop(0, (NCH - 6) // 4, _quad, 0)

    _handle(NCH - 4, 0, 2, True, True, True)
    _handle(NCH - 3, 1, 3, True, True, True)
    _handle(NCH - 2, 0, 0, True, False, True)
    _handle(NCH - 1, 1, 1, True, False, False)
    _sc_drain(0, 0)
    _sc_drain(1, 1)

    plsc.subcore_barrier()
    pltpu.sync_copy(agg.at[pl.ds(base, ROWS_PT)],
                    out_hbm.at[cid, pl.ds(base, ROWS_PT)])

    @pl.when(sid == NS - 1)
    def _flush_tail():
        pltpu.sync_copy(agg.at[pl.ds(NS * ROWS_PT, N - NS * ROWS_PT)],
                        out_hbm.at[cid, pl.ds(NS * ROWS_PT, N - NS * ROWS_PT)])


_sc_edges = functools.partial(
    pl.kernel,
    out_type=jax.ShapeDtypeStruct((NC, N, D), jnp.float32),
    mesh=plsc.VectorSubcoreMesh(core_axis_name="c", subcore_axis_name="s"),
    scratch_types=[
        pltpu.VMEM((2, CHUNK), jnp.int32),
        pltpu.VMEM((2, CHUNK), jnp.int32),
        pltpu.VMEM((2, CHUNK), jnp.int32),
        pltpu.VMEM((2, CHUNK), jnp.int32),
        pltpu.VMEM((CHUNK, D), jnp.int32),
        pltpu.VMEM((CHUNK, D), jnp.int32),
        pltpu.VMEM((CHUNK, HD), jnp.int32),
        pltpu.VMEM((CHUNK, D), jnp.float32),
        pltpu.VMEM((CHUNK, D), jnp.int32),
        pltpu.VMEM((CHUNK, D), jnp.int32),
        pltpu.VMEM((CHUNK, HD), jnp.int32),
        pltpu.VMEM((CHUNK, D), jnp.float32),
        pltpu.VMEM_SHARED((N, D), jnp.float32),
        pltpu.SemaphoreType.DMA,
        pltpu.SemaphoreType.DMA,
        pltpu.SemaphoreType.DMA,
        pltpu.SemaphoreType.DMA,
        pltpu.SemaphoreType.DMA,
        pltpu.SemaphoreType.DMA,
        pltpu.SemaphoreType.DMA,
        pltpu.SemaphoreType.DMA,
    ],
)(_sc_edges_body)


def kernel(x, edge_index, edge_attr, W_msg, b_msg, W_upd, b_upd):
    tnode = pl.pallas_call(
        _proj_body,
        grid=(N // RB,),
        in_specs=[
            pl.BlockSpec((RB, D), lambda i: (i, 0)),
            pl.BlockSpec((D, D), lambda i: (0, 0)),
            pl.BlockSpec((D, D), lambda i: (1, 0)),
            pl.BlockSpec((D,), lambda i: (0,)),
        ],
        out_specs=pl.BlockSpec((RB, D), lambda i: (i, 0)),
        out_shape=jax.ShapeDtypeStruct((N, D), jnp.int32),
    )(x, W_msg, W_msg, b_msg)

    c_edge = pl.pallas_call(
        _edge_term_body,
        grid=(E // EB,),
        in_specs=[
            pl.BlockSpec((EB, DE), lambda i: (i, 0)),
            pl.BlockSpec((8, D), lambda i: (2 * D // 8, 0)),
        ],
        out_specs=pl.BlockSpec((EB, HD), lambda i: (i, 0)),
        out_shape=jax.ShapeDtypeStruct((E, HD), jnp.int32),
    )(edge_attr, W_msg)

    src1d, dst1d = pl.pallas_call(
        _split_body,
        grid=(E // EI,),
        in_specs=[pl.BlockSpec((2, EI), lambda i: (0, i))],
        out_specs=[
            pl.BlockSpec((EI,), lambda i: (i,)),
            pl.BlockSpec((EI,), lambda i: (i,)),
        ],
        out_shape=[
            jax.ShapeDtypeStruct((E,), jnp.int32),
            jax.ShapeDtypeStruct((E,), jnp.int32),
        ],
    )(edge_index)

    parts = _sc_edges(tnode, src1d, dst1d,
                      c_edge.reshape(NW * NCH, CHUNK, HD))

    out = pl.pallas_call(
        _update_body,
        grid=(N // RB,),
        in_specs=[
            pl.BlockSpec((RB, D), lambda i: (i, 0)),
            pl.BlockSpec((NC, RB, D), lambda i: (0, i, 0)),
            pl.BlockSpec((D, D), lambda i: (0, 0)),
            pl.BlockSpec((D, D), lambda i: (1, 0)),
            pl.BlockSpec((D,), lambda i: (0,)),
        ],
        out_specs=pl.BlockSpec((RB, D), lambda i: (i, 0)),
        out_shape=jax.ShapeDtypeStruct((N, D), jnp.float32),
    )(x, parts, W_upd, W_upd, b_upd)
    return out


# speedup vs baseline: 1.0479x; 1.0324x over previous
import functools

import jax
import jax.numpy as jnp
from jax import lax
from jax.experimental import pallas as pl
from jax.experimental.pallas import tpu as pltpu
from jax.experimental.pallas import tpu_sc as plsc

N = 10000
E = 320000
D = 128
DE = 4

NC = 2
NS = 16
NW = NC * NS
EPT = E // NW
CHUNK = 40
NCH = EPT // CHUNK
ROWS_PT = 624
RB = 1000
EB = 8000
EI = E
HD = D // 2

FP_SCALE = 1024.0
FP_BIAS = 8192


def _pack_pairs(p):
    q = jnp.clip(jnp.round(p * FP_SCALE), -8191.0, 8191.0) + float(FP_BIAS)
    qi = q.astype(jnp.int32)
    return (qi[:, HD:] << 16) | qi[:, :HD]


def _proj_body(x_ref, w1_ref, w2_ref, b_ref, p1_ref, p2_ref):
    xb = x_ref[...]
    p1_ref[...] = (jnp.dot(xb, w1_ref[...], preferred_element_type=jnp.float32)
                   + b_ref[...][None, :])
    p2_ref[...] = jnp.dot(xb, w2_ref[...], preferred_element_type=jnp.float32)


def _edge_term_body(ea_ref, w3_ref, c_ref):
    c_ref[...] = jnp.dot(ea_ref[...], w3_ref[0:DE, :],
                         preferred_element_type=jnp.float32)


def _split_body(ei_ref, s_ref, d_ref):
    s_ref[...] = ei_ref[0]
    d_ref[...] = ei_ref[1]


def _update_body(x_ref, p_ref, wu1_ref, wu2_ref, b_ref, o_ref):
    xb = x_ref[...]
    agg = p_ref[0] + p_ref[1]
    h = (jnp.dot(xb, wu1_ref[...], preferred_element_type=jnp.float32)
         + jnp.dot(agg, wu2_ref[...], preferred_element_type=jnp.float32)
         + b_ref[...][None, :])
    o_ref[...] = xb + jnp.maximum(h, 0.0)


def _sc_edges_body(p1_hbm, p2_hbm, src_hbm, dst_hbm, c_hbm, out_hbm,
                   i0, i1, a0, b0, c0, a1, b1, c1, agg,
                   sg0, sg1, si0, si1):
    cid = lax.axis_index("c")
    sid = lax.axis_index("s")
    wid = cid * NS + sid

    def _zrow(r, carry):
        for d in range(D // 16):
            a0[r, pl.ds(d * 16, 16)] = jnp.zeros((16,), jnp.float32)
        return carry
    lax.fori_loop(0, CHUNK, _zrow, 0)
    base = sid * ROWS_PT
    for k in range(ROWS_PT // CHUNK):
        pltpu.sync_copy(a0, agg.at[pl.ds(base + k * CHUNK, CHUNK)])
    rem = ROWS_PT % CHUNK
    if rem:
        pltpu.sync_copy(a0.at[pl.ds(0, rem)],
                        agg.at[pl.ds(base + ROWS_PT - rem, rem)])

    @pl.when(sid == NS - 1)
    def _zero_tail():
        pltpu.sync_copy(a0.at[pl.ds(0, N - NS * ROWS_PT)],
                        agg.at[pl.ds(NS * ROWS_PT, N - NS * ROWS_PT)])
    plsc.subcore_barrier()

    slots = ((i0, a0, b0, c0, sg0, si0), (i1, a1, b1, c1, sg1, si1))
    ibase = wid * EPT

    def _process(bufA, bufB, bufC):
        def _edge(e, carry):
            for d in range(D // 16):
                sl = pl.ds(d * 16, 16)
                v = bufA[e, sl] + bufB[e, sl] + bufC[e, sl]
                bufA[e, sl] = jnp.maximum(v, 0.0)
            return carry
        lax.fori_loop(0, CHUNK, _edge, 0)

    def _idx_issue(j, iv, si):
        pltpu.async_copy(src_hbm.at[pl.ds(ibase + j * CHUNK, CHUNK)],
                         iv.at[0], si)
        pltpu.async_copy(dst_hbm.at[pl.ds(ibase + j * CHUNK, CHUNK)],
                         iv.at[1], si)

    def _idx_drain(iv, si):
        pltpu.make_async_copy(src_hbm.at[pl.ds(0, CHUNK)],
                              iv.at[0], si).wait()
        pltpu.make_async_copy(src_hbm.at[pl.ds(0, CHUNK)],
                              iv.at[1], si).wait()

    def _issue_gathers(j, s):
        iv, bufA, bufB, bufC, sg, _ = slots[s]
        pltpu.async_copy(p1_hbm.at[iv.at[0]], bufA, sg)
        pltpu.async_copy(p2_hbm.at[iv.at[1]], bufB, sg)
        pltpu.async_copy(c_hbm.at[wid * NCH + j], bufC, sg)

    def _handle(j, s, prefetch_idx, issue_next):
        iv, bufA, bufB, bufC, sg, si = slots[s]
        ivn, _, _, _, _, sin = slots[1 - s]
        pltpu.make_async_copy(p1_hbm.at[iv.at[0]], bufA, sg).wait()
        pltpu.make_async_copy(p2_hbm.at[iv.at[1]], bufB, sg).wait()
        pltpu.make_async_copy(c_hbm.at[0], bufC, sg).wait()
        if issue_next:
            _idx_drain(ivn, sin)
            _issue_gathers(j + 1, 1 - s)
        _process(bufA, bufB, bufC)
        pltpu.sync_copy(bufA, agg.at[iv.at[1]], add=True)
        if prefetch_idx:
            _idx_issue(j + 2, iv, si)

    _idx_issue(0, i0, si0)
    _idx_drain(i0, si0)
    _issue_gathers(0, 0)
    _idx_issue(1, i1, si1)

    def _pair(i, carry):
        j0 = 2 * i
        _handle(j0, 0, True, True)
        _handle(j0 + 1, 1, True, True)
        return carry
    lax.fori_loop(0, NCH // 2 - 1, _pair, 0)

    _handle(NCH - 2, 0, False, True)
    _handle(NCH - 1, 1, False, False)

    plsc.subcore_barrier()
    pltpu.sync_copy(agg.at[pl.ds(base, ROWS_PT)],
                    out_hbm.at[cid, pl.ds(base, ROWS_PT)])

    @pl.when(sid == NS - 1)
    def _flush_tail():
        pltpu.sync_copy(agg.at[pl.ds(NS * ROWS_PT, N - NS * ROWS_PT)],
                        out_hbm.at[cid, pl.ds(NS * ROWS_PT, N - NS * ROWS_PT)])


_sc_edges = functools.partial(
    pl.kernel,
    out_type=jax.ShapeDtypeStruct((NC, N, D), jnp.float32),
    mesh=plsc.VectorSubcoreMesh(core_axis_name="c", subcore_axis_name="s"),
    scratch_types=[
        pltpu.VMEM((2, CHUNK), jnp.int32),
        pltpu.VMEM((2, CHUNK), jnp.int32),
        pltpu.VMEM((CHUNK, D), jnp.float32),
        pltpu.VMEM((CHUNK, D), jnp.float32),
        pltpu.VMEM((CHUNK, D), jnp.float32),
        pltpu.VMEM((CHUNK, D), jnp.float32),
        pltpu.VMEM((CHUNK, D), jnp.float32),
        pltpu.VMEM((CHUNK, D), jnp.float32),
        pltpu.VMEM_SHARED((N, D), jnp.float32),
        pltpu.SemaphoreType.DMA,
        pltpu.SemaphoreType.DMA,
        pltpu.SemaphoreType.DMA,
        pltpu.SemaphoreType.DMA,
    ],
)(_sc_edges_body)


def kernel(x, edge_index, edge_attr, W_msg, b_msg, W_upd, b_upd):
    p1, p2 = pl.pallas_call(
        _proj_body,
        grid=(N // RB,),
        in_specs=[
            pl.BlockSpec((RB, D), lambda i: (i, 0)),
            pl.BlockSpec((D, D), lambda i: (0, 0)),
            pl.BlockSpec((D, D), lambda i: (1, 0)),
            pl.BlockSpec((D,), lambda i: (0,)),
        ],
        out_specs=[
            pl.BlockSpec((RB, D), lambda i: (i, 0)),
            pl.BlockSpec((RB, D), lambda i: (i, 0)),
        ],
        out_shape=[
            jax.ShapeDtypeStruct((N, D), jnp.float32),
            jax.ShapeDtypeStruct((N, D), jnp.float32),
        ],
    )(x, W_msg, W_msg, b_msg)

    c_edge = pl.pallas_call(
        _edge_term_body,
        grid=(E // EB,),
        in_specs=[
            pl.BlockSpec((EB, DE), lambda i: (i, 0)),
            pl.BlockSpec((8, D), lambda i: (2 * D // 8, 0)),
        ],
        out_specs=pl.BlockSpec((EB, D), lambda i: (i, 0)),
        out_shape=jax.ShapeDtypeStruct((E, D), jnp.float32),
    )(edge_attr, W_msg)

    src1d, dst1d = pl.pallas_call(
        _split_body,
        grid=(E // EI,),
        in_specs=[pl.BlockSpec((2, EI), lambda i: (0, i))],
        out_specs=[
            pl.BlockSpec((EI,), lambda i: (i,)),
            pl.BlockSpec((EI,), lambda i: (i,)),
        ],
        out_shape=[
            jax.ShapeDtypeStruct((E,), jnp.int32),
            jax.ShapeDtypeStruct((E,), jnp.int32),
        ],
    )(edge_index)

    parts = _sc_edges(p1, p2, src1d, dst1d,
                      c_edge.reshape(NW * NCH, CHUNK, D))

    out = pl.pallas_call(
        _update_body,
        grid=(N // RB,),
        in_specs=[
            pl.BlockSpec((RB, D), lambda i: (i, 0)),
            pl.BlockSpec((NC, RB, D), lambda i: (0, i, 0)),
            pl.BlockSpec((D, D), lambda i: (0, 0)),
            pl.BlockSpec((D, D), lambda i: (1, 0)),
            pl.BlockSpec((D,), lambda i: (0,)),
        ],
        out_specs=pl.BlockSpec((RB, D), lambda i: (i, 0)),
        out_shape=jax.ShapeDtypeStruct((N, D), jnp.float32),
    )(x, parts, W_upd, W_upd, b_upd)
    return out


# speedup vs baseline: 1.0481x; 1.0002x over previous
import functools

import jax
import jax.numpy as jnp
from jax import lax
from jax.experimental import pallas as pl
from jax.experimental.pallas import tpu as pltpu
from jax.experimental.pallas import tpu_sc as plsc

N = 10000
E = 320000
D = 128
DE = 4

NC = 2
NS = 16
NW = NC * NS
EPT = E // NW
CHUNK = 40
NCH = EPT // CHUNK
ROWS_PT = 624
RB = 1000
EB = 8000
EI = E
HD = D // 2
EH = E // 2
EPTH = EH // NW
NCHH = EPTH // CHUNK

FP_SCALE = 1024.0
FP_BIAS = 8192


def _pack_pairs(p):
    q = jnp.clip(jnp.round(p * FP_SCALE), -8191.0, 8191.0) + float(FP_BIAS)
    qi = q.astype(jnp.int32)
    return (qi[:, HD:] << 16) | qi[:, :HD]


def _proj_body(x_ref, w1_ref, w2_ref, b_ref, p1_ref, p2_ref):
    xb = x_ref[...]
    p1_ref[...] = (jnp.dot(xb, w1_ref[...], preferred_element_type=jnp.float32)
                   + b_ref[...][None, :])
    p2_ref[...] = jnp.dot(xb, w2_ref[...], preferred_element_type=jnp.float32)


def _edge_term_body(ea_ref, w3_ref, c_ref):
    c_ref[...] = jnp.dot(ea_ref[...], w3_ref[0:DE, :],
                         preferred_element_type=jnp.float32)


def _split_body(ei_ref, s_ref, d_ref):
    s_ref[...] = ei_ref[0]
    d_ref[...] = ei_ref[1]


def _update_body(x_ref, p_ref, q_ref, wu1_ref, wu2_ref, b_ref, o_ref):
    xb = x_ref[...]
    agg = (p_ref[0] + p_ref[1]) + (q_ref[0] + q_ref[1])
    h = (jnp.dot(xb, wu1_ref[...], preferred_element_type=jnp.float32)
         + jnp.dot(agg, wu2_ref[...], preferred_element_type=jnp.float32)
         + b_ref[...][None, :])
    o_ref[...] = xb + jnp.maximum(h, 0.0)


def _sc_edges_body(half, p1_hbm, p2_hbm, src_hbm, dst_hbm, c_hbm, out_hbm,
                   i0, i1, a0, b0, c0, a1, b1, c1, agg,
                   sg0, sg1, si0, si1):
    cid = lax.axis_index("c")
    sid = lax.axis_index("s")
    wid = cid * NS + sid

    def _zrow(r, carry):
        for d in range(D // 16):
            a0[r, pl.ds(d * 16, 16)] = jnp.zeros((16,), jnp.float32)
        return carry
    lax.fori_loop(0, CHUNK, _zrow, 0)
    base = sid * ROWS_PT
    for k in range(ROWS_PT // CHUNK):
        pltpu.sync_copy(a0, agg.at[pl.ds(base + k * CHUNK, CHUNK)])
    rem = ROWS_PT % CHUNK
    if rem:
        pltpu.sync_copy(a0.at[pl.ds(0, rem)],
                        agg.at[pl.ds(base + ROWS_PT - rem, rem)])

    @pl.when(sid == NS - 1)
    def _zero_tail():
        pltpu.sync_copy(a0.at[pl.ds(0, N - NS * ROWS_PT)],
                        agg.at[pl.ds(NS * ROWS_PT, N - NS * ROWS_PT)])
    plsc.subcore_barrier()

    slots = ((i0, a0, b0, c0, sg0, si0), (i1, a1, b1, c1, sg1, si1))
    ibase = half * EH + wid * EPTH

    def _process(bufA, bufB, bufC):
        def _edge(e, carry):
            for d in range(D // 16):
                sl = pl.ds(d * 16, 16)
                v = bufA[e, sl] + bufB[e, sl] + bufC[e, sl]
                bufA[e, sl] = jnp.maximum(v, 0.0)
            return carry
        lax.fori_loop(0, CHUNK, _edge, 0)

    def _idx_issue(j, iv, si):
        pltpu.async_copy(src_hbm.at[pl.ds(ibase + j * CHUNK, CHUNK)],
                         iv.at[0], si)
        pltpu.async_copy(dst_hbm.at[pl.ds(ibase + j * CHUNK, CHUNK)],
                         iv.at[1], si)

    def _idx_drain(iv, si):
        pltpu.make_async_copy(src_hbm.at[pl.ds(0, CHUNK)],
                              iv.at[0], si).wait()
        pltpu.make_async_copy(src_hbm.at[pl.ds(0, CHUNK)],
                              iv.at[1], si).wait()

    def _issue_gathers(j, s):
        iv, bufA, bufB, bufC, sg, _ = slots[s]
        pltpu.async_copy(p1_hbm.at[iv.at[0]], bufA, sg)
        pltpu.async_copy(p2_hbm.at[iv.at[1]], bufB, sg)
        pltpu.async_copy(c_hbm.at[wid * NCHH + j], bufC, sg)

    def _handle(j, s, prefetch_idx, issue_next):
        iv, bufA, bufB, bufC, sg, si = slots[s]
        ivn, _, _, _, _, sin = slots[1 - s]
        pltpu.make_async_copy(p1_hbm.at[iv.at[0]], bufA, sg).wait()
        pltpu.make_async_copy(p2_hbm.at[iv.at[1]], bufB, sg).wait()
        pltpu.make_async_copy(c_hbm.at[0], bufC, sg).wait()
        if issue_next:
            _idx_drain(ivn, sin)
            _issue_gathers(j + 1, 1 - s)
        _process(bufA, bufB, bufC)
        pltpu.sync_copy(bufA, agg.at[iv.at[1]], add=True)
        if prefetch_idx:
            _idx_issue(j + 2, iv, si)

    _idx_issue(0, i0, si0)
    _idx_drain(i0, si0)
    _issue_gathers(0, 0)
    _idx_issue(1, i1, si1)

    def _pair(i, carry):
        j0 = 2 * i
        _handle(j0, 0, True, True)
        _handle(j0 + 1, 1, True, True)
        return carry
    lax.fori_loop(0, (NCHH - 3) // 2, _pair, 0)

    _handle(NCHH - 3, 0, True, True)
    _handle(NCHH - 2, 1, False, True)
    _handle(NCHH - 1, 0, False, False)

    plsc.subcore_barrier()
    pltpu.sync_copy(agg.at[pl.ds(base, ROWS_PT)],
                    out_hbm.at[cid, pl.ds(base, ROWS_PT)])

    @pl.when(sid == NS - 1)
    def _flush_tail():
        pltpu.sync_copy(agg.at[pl.ds(NS * ROWS_PT, N - NS * ROWS_PT)],
                        out_hbm.at[cid, pl.ds(NS * ROWS_PT, N - NS * ROWS_PT)])


def _make_sc_edges(half):
  return functools.partial(
    pl.kernel,
    out_type=jax.ShapeDtypeStruct((NC, N, D), jnp.float32),
    mesh=plsc.VectorSubcoreMesh(core_axis_name="c", subcore_axis_name="s"),
    scratch_types=[
        pltpu.VMEM((2, CHUNK), jnp.int32),
        pltpu.VMEM((2, CHUNK), jnp.int32),
        pltpu.VMEM((CHUNK, D), jnp.float32),
        pltpu.VMEM((CHUNK, D), jnp.float32),
        pltpu.VMEM((CHUNK, D), jnp.float32),
        pltpu.VMEM((CHUNK, D), jnp.float32),
        pltpu.VMEM((CHUNK, D), jnp.float32),
        pltpu.VMEM((CHUNK, D), jnp.float32),
        pltpu.VMEM_SHARED((N, D), jnp.float32),
        pltpu.SemaphoreType.DMA,
        pltpu.SemaphoreType.DMA,
        pltpu.SemaphoreType.DMA,
        pltpu.SemaphoreType.DMA,
    ],
  )(functools.partial(_sc_edges_body, half))


_sc_edges_h0 = _make_sc_edges(0)
_sc_edges_h1 = _make_sc_edges(1)


def kernel(x, edge_index, edge_attr, W_msg, b_msg, W_upd, b_upd):
    p1, p2 = pl.pallas_call(
        _proj_body,
        grid=(N // RB,),
        in_specs=[
            pl.BlockSpec((RB, D), lambda i: (i, 0)),
            pl.BlockSpec((D, D), lambda i: (0, 0)),
            pl.BlockSpec((D, D), lambda i: (1, 0)),
            pl.BlockSpec((D,), lambda i: (0,)),
        ],
        out_specs=[
            pl.BlockSpec((RB, D), lambda i: (i, 0)),
            pl.BlockSpec((RB, D), lambda i: (i, 0)),
        ],
        out_shape=[
            jax.ShapeDtypeStruct((N, D), jnp.float32),
            jax.ShapeDtypeStruct((N, D), jnp.float32),
        ],
    )(x, W_msg, W_msg, b_msg)

    def _c_half(h):
        return pl.pallas_call(
            _edge_term_body,
            grid=(EH // EB,),
            in_specs=[
                pl.BlockSpec((EB, DE), lambda i, h=h: (i + h * (EH // EB), 0)),
                pl.BlockSpec((8, D), lambda i: (2 * D // 8, 0)),
            ],
            out_specs=pl.BlockSpec((EB, D), lambda i: (i, 0)),
            out_shape=jax.ShapeDtypeStruct((EH, D), jnp.float32),
        )(edge_attr, W_msg)

    c0_edge = _c_half(0)
    c1_edge = _c_half(1)

    src1d, dst1d = pl.pallas_call(
        _split_body,
        grid=(E // EI,),
        in_specs=[pl.BlockSpec((2, EI), lambda i: (0, i))],
        out_specs=[
            pl.BlockSpec((EI,), lambda i: (i,)),
            pl.BlockSpec((EI,), lambda i: (i,)),
        ],
        out_shape=[
            jax.ShapeDtypeStruct((E,), jnp.int32),
            jax.ShapeDtypeStruct((E,), jnp.int32),
        ],
    )(edge_index)

    parts0 = _sc_edges_h0(p1, p2, src1d, dst1d,
                          c0_edge.reshape(NW * NCHH, CHUNK, D))
    parts1 = _sc_edges_h1(p1, p2, src1d, dst1d,
                          c1_edge.reshape(NW * NCHH, CHUNK, D))

    out = pl.pallas_call(
        _update_body,
        grid=(N // RB,),
        in_specs=[
            pl.BlockSpec((RB, D), lambda i: (i, 0)),
            pl.BlockSpec((NC, RB, D), lambda i: (0, i, 0)),
            pl.BlockSpec((NC, RB, D), lambda i: (0, i, 0)),
            pl.BlockSpec((D, D), lambda i: (0, 0)),
            pl.BlockSpec((D, D), lambda i: (1, 0)),
            pl.BlockSpec((D,), lambda i: (0,)),
        ],
        out_specs=pl.BlockSpec((RB, D), lambda i: (i, 0)),
        out_shape=jax.ShapeDtypeStruct((N, D), jnp.float32),
    )(x, parts0, parts1, W_upd, W_upd, b_upd)
    return out
